# baseline (speedup 1.0000x reference)
import functools

import jax
import jax.numpy as jnp
from jax import lax
from jax.experimental import pallas as pl
from jax.experimental.pallas import tpu as pltpu
from jax.experimental.pallas import tpu_sc as plsc

_N = 10000
_E = 320000
_D = 128
_DE = 16

_NC = 2
_NS = 16
_EC = _E // _NC
_ET = _EC // _NS
_CH = 40
_NCH = _ET // _CH
_CH8 = _CH // 8
_ZR = (_N // _NS) // 8 * 8


def _node_tables_body(x_ref, ws_ref, wr_ref, bs_ref, wsm_ref, wdm_ref,
                      xw_ref, xr_ref, xs_ref, xd_ref):
    xb = x_ref[...]
    xw_ref[...] = jnp.dot(xb, ws_ref[...], preferred_element_type=jnp.float32)
    xr_ref[...] = jnp.dot(xb, wr_ref[...], preferred_element_type=jnp.float32) + bs_ref[...]
    xs_ref[...] = jnp.dot(xb, wsm_ref[...], preferred_element_type=jnp.float32)
    xd_ref[...] = jnp.dot(xb, wdm_ref[...], preferred_element_type=jnp.float32)


def _node_tables(x, W_self, W_root, b_self2d, W_s, W_d):
    bn = 2000
    grid = _N // bn
    full = lambda shape: pl.BlockSpec(shape, lambda i: (0, 0))
    return pl.pallas_call(
        _node_tables_body,
        grid=(grid,),
        in_specs=[
            pl.BlockSpec((bn, _D), lambda i: (i, 0)),
            full((_D, _D)), full((_D, _D)), full((1, _D)),
            full((_D, _DE)), full((_D, _DE)),
        ],
        out_specs=[
            pl.BlockSpec((bn, _D), lambda i: (i, 0)),
            pl.BlockSpec((bn, _D), lambda i: (i, 0)),
            pl.BlockSpec((bn, _DE), lambda i: (i, 0)),
            pl.BlockSpec((bn, _DE), lambda i: (i, 0)),
        ],
        out_shape=[
            jax.ShapeDtypeStruct((_N, _D), jnp.float32),
            jax.ShapeDtypeStruct((_N, _D), jnp.float32),
            jax.ShapeDtypeStruct((_N, _DE), jnp.float32),
            jax.ShapeDtypeStruct((_N, _DE), jnp.float32),
        ],
    )(x, W_self, W_root, b_self2d, W_s, W_d)


def _edge_tables_body(e8_ref, wa_ref, be_ref, wg_ref, bg_ref, ea_ref, gate_ref):
    eb = e8_ref[...]
    ea_ref[...] = jnp.dot(eb, wa_ref[...], preferred_element_type=jnp.float32) + be_ref[...]
    z = jnp.dot(eb, wg_ref[...], preferred_element_type=jnp.float32) + bg_ref[...]
    gate_ref[...] = jax.nn.sigmoid(z)


def _edge_tables(edge_attr, W_a, b_e, w_gate, b_gate):
    e8 = edge_attr.reshape(_E // 8, 128)
    eye8 = jnp.eye(8, dtype=jnp.float32)
    wa_bd = jnp.einsum("ab,cd->acbd", eye8, W_a).reshape(128, 128)
    wg_bd = jnp.einsum("ab,cd->acbd", eye8, w_gate).reshape(128, 8)
    be_bd = jnp.tile(b_e, 8).reshape(1, 128)
    bg2d = b_gate.reshape(1, 1)
    be8 = 4000
    grid = (_E // 8) // be8
    full = lambda shape: pl.BlockSpec(shape, lambda i: (0, 0))
    ea8, gate8 = pl.pallas_call(
        _edge_tables_body,
        grid=(grid,),
        in_specs=[
            pl.BlockSpec((be8, 128), lambda i: (i, 0)),
            full((128, 128)), full((1, 128)), full((128, 8)), full((1, 1)),
        ],
        out_specs=[
            pl.BlockSpec((be8, 128), lambda i: (i, 0)),
            pl.BlockSpec((be8, 8), lambda i: (i, 0)),
        ],
        out_shape=[
            jax.ShapeDtypeStruct((_E // 8, 128), jnp.float32),
            jax.ShapeDtypeStruct((_E // 8, 8), jnp.float32),
        ],
    )(e8, wa_bd, be_bd, wg_bd, bg2d)
    return ea8, gate8


@functools.cache
def _make_sc_sparse():
    mesh = plsc.VectorSubcoreMesh(core_axis_name="c", subcore_axis_name="s",
                                  num_cores=_NC, num_subcores=_NS)
    return pl.kernel(
        _sc_sparse_body,
        mesh=mesh,
        out_type=[
            jax.ShapeDtypeStruct((_NC * _N, _D), jnp.float32),
            jax.ShapeDtypeStruct((_E, _DE), jnp.float32),
        ],
        scratch_types=[
            pltpu.VMEM_SHARED((_N, _D), jnp.float32),
            pltpu.VMEM((_ET,), jnp.int32),
            pltpu.VMEM((_ET,), jnp.int32),
            pltpu.VMEM((_CH,), jnp.float32),
            pltpu.VMEM((_CH,), jnp.float32),
            pltpu.VMEM((_CH,), jnp.int32),
            pltpu.VMEM((_CH,), jnp.int32),
            pltpu.VMEM((_CH, _D), jnp.float32),
            pltpu.VMEM((_CH, _D), jnp.float32),
            pltpu.VMEM((_CH, _D), jnp.float32),
            pltpu.VMEM((_CH, _D), jnp.float32),
            pltpu.VMEM((_CH, _DE), jnp.float32),
            pltpu.VMEM((_CH, _DE), jnp.float32),
            pltpu.VMEM((_CH, _DE), jnp.float32),
            pltpu.VMEM((_CH, _DE), jnp.float32),
            pltpu.VMEM((_CH8, 128), jnp.float32),
            pltpu.VMEM((_CH8, 128), jnp.float32),
            pltpu.VMEM((_CH, _DE), jnp.float32),
            pltpu.VMEM((_CH, _DE), jnp.float32),
            pltpu.SemaphoreType.DMA,
            pltpu.SemaphoreType.DMA,
            pltpu.SemaphoreType.DMA,
            pltpu.SemaphoreType.DMA,
            pltpu.SemaphoreType.DMA,
            pltpu.SemaphoreType.DMA,
            pltpu.SemaphoreType.DMA,
        ],
        compiler_params=pltpu.CompilerParams(use_tc_tiling_on_sc=False),
    )


def _sc_sparse_body(src_hbm, dst_hbm, gate_hbm, xw_hbm, xs_hbm, xd_hbm,
                    ea8_hbm, agg_out, he_out,
                    aggs, srcall, dstall, gv0, gv1, dv0, dv1,
                    rows0, rows1, scat0, scat1, xs0, xs1, xd0, xd1,
                    ea0, ea1, he0, he1,
                    gsem0, gsem1, ssem0, ssem1, hsem0, hsem1, stsem):
    cid = lax.axis_index("c")
    sid = lax.axis_index("s")
    gv = (gv0, gv1)
    dv = (dv0, dv1)
    rows = (rows0, rows1)
    scat = (scat0, scat1)
    xsv = (xs0, xs1)
    xdv = (xd0, xd1)
    eav = (ea0, ea1)
    hev = (he0, he1)
    gsem = (gsem0, gsem1)
    ssem = (ssem0, ssem1)
    hsem = (hsem0, hsem1)
    zero16 = jnp.zeros((16,), jnp.float32)

    tile_base = cid * _EC + sid * _ET
    tile8 = tile_base // 8
    row0 = sid * _ZR

    pltpu.async_copy(src_hbm.at[pl.ds(tile_base, _ET)], srcall, stsem)
    pltpu.async_copy(dst_hbm.at[pl.ds(tile_base, _ET)], dstall, stsem)

    for r in range(_CH):
        for j in range(_D // 16):
            rows0[r, pl.ds(16 * j, 16)] = zero16
    for i in range(_ZR // _CH):
        pltpu.sync_copy(rows0, aggs.at[pl.ds(row0 + i * _CH, _CH)])
    rem = _ZR % _CH
    if rem:
        pltpu.sync_copy(rows0.at[pl.ds(0, rem)],
                        aggs.at[pl.ds(row0 + (_ZR // _CH) * _CH, rem)])
    @pl.when(sid == _NS - 1)
    def _zero_tail():
        pltpu.sync_copy(rows0.at[pl.ds(0, _N - _NS * _ZR)],
                        aggs.at[pl.ds(_NS * _ZR, _N - _NS * _ZR)])
    pltpu.make_async_copy(src_hbm.at[pl.ds(tile_base, _ET)], srcall, stsem).wait()
    pltpu.make_async_copy(dst_hbm.at[pl.ds(tile_base, _ET)], dstall, stsem).wait()
    plsc.subcore_barrier()

    def issue_gathers(b, c):
        base = tile_base + c * _CH
        sidx = srcall.at[pl.ds(c * _CH, _CH)]
        didx = dstall.at[pl.ds(c * _CH, _CH)]
        pltpu.async_copy(xw_hbm.at[sidx], rows[b], gsem[b])
        pltpu.async_copy(xs_hbm.at[sidx], xsv[b], gsem[b])
        pltpu.async_copy(xd_hbm.at[didx], xdv[b], gsem[b])
        pltpu.async_copy(ea8_hbm.at[pl.ds(tile8 + c * _CH8, _CH8)], eav[b],
                         gsem[b])
        pltpu.async_copy(gate_hbm.at[pl.ds(base, _CH)], gv[b], gsem[b])

    def wait_gathers(b, c):
        base = tile_base + c * _CH
        sidx = srcall.at[pl.ds(c * _CH, _CH)]
        didx = dstall.at[pl.ds(c * _CH, _CH)]
        pltpu.make_async_copy(xw_hbm.at[sidx], rows[b], gsem[b]).wait()
        pltpu.make_async_copy(xs_hbm.at[sidx], xsv[b], gsem[b]).wait()
        pltpu.make_async_copy(xd_hbm.at[didx], xdv[b], gsem[b]).wait()
        pltpu.make_async_copy(ea8_hbm.at[pl.ds(tile8 + c * _CH8, _CH8)],
                              eav[b], gsem[b]).wait()
        pltpu.make_async_copy(gate_hbm.at[pl.ds(base, _CH)], gv[b], gsem[b]).wait()

    def wait_scatter(b, c):
        pltpu.make_async_copy(scat[b], aggs.at[dv[b]], ssem[b]).wait()

    def wait_hestore(b, c):
        pltpu.make_async_copy(hev[b],
                              he_out.at[pl.ds(tile_base + c * _CH, _CH)],
                              hsem[b]).wait()

    def process(b, c, first):
        wait_gathers(b, c)
        if not first:
            wait_scatter(b, c)
            wait_hestore(b, c)
        dv[b][pl.ds(0, 16)] = dstall[pl.ds(c * _CH, 16)]
        dv[b][pl.ds(16, 16)] = dstall[pl.ds(c * _CH + 16, 16)]
        dv[b][pl.ds(24, 16)] = dstall[pl.ds(c * _CH + 24, 16)]
        g0 = gv[b][pl.ds(0, 16)]
        g1 = gv[b][pl.ds(16, 16)]
        g2 = gv[b][pl.ds(24, 16)]
        for e in range(_CH):
            if e < 16:
                g = g0[e]
            elif e < 32:
                g = g1[e - 16]
            else:
                g = g2[e - 24]
            for j in range(_D // 16):
                sl = pl.ds(16 * j, 16)
                scat[b][e, sl] = rows[b][e, sl] * g
        pltpu.async_copy(scat[b], aggs.at[dv[b]], ssem[b], add=True)
        for e in range(_CH):
            sl = pl.ds((e % 8) * 16, 16)
            hev[b][e, :] = jnp.maximum(
                xsv[b][e, :] + xdv[b][e, :] + eav[b][e // 8, sl], 0.0)
        pltpu.async_copy(hev[b], he_out.at[pl.ds(tile_base + c * _CH, _CH)],
                         hsem[b])

    issue_gathers(0, 0)
    issue_gathers(1, 1)

    def pair_body(i, carry):
        c0 = 2 * i

        @pl.when(i == 0)
        def _first():
            process(0, c0, True)
            issue_gathers(0, c0 + 2)
            process(1, c0 + 1, True)
            issue_gathers(1, c0 + 3)

        @pl.when(i > 0)
        def _steady():
            process(0, c0, False)
            issue_gathers(0, c0 + 2)
            process(1, c0 + 1, False)
            issue_gathers(1, c0 + 3)

        return carry

    lax.fori_loop(0, _NCH // 2 - 1, pair_body, 0)
    process(0, _NCH - 2, False)
    process(1, _NCH - 1, False)
    wait_scatter(0, _NCH - 2)
    wait_hestore(0, _NCH - 2)
    wait_scatter(1, _NCH - 1)
    wait_hestore(1, _NCH - 1)

    plsc.subcore_barrier()
    pltpu.sync_copy(aggs.at[pl.ds(row0, _ZR)],
                    agg_out.at[pl.ds(cid * _N + row0, _ZR)])
    @pl.when(sid == _NS - 1)
    def _flush_tail():
        pltpu.sync_copy(aggs.at[pl.ds(_NS * _ZR, _N - _NS * _ZR)],
                        agg_out.at[pl.ds(cid * _N + _NS * _ZR, _N - _NS * _ZR)])


def _finalize_body(a0_ref, a1_ref, xr_ref, hv_ref):
    hv_ref[...] = jnp.maximum(a0_ref[...] + a1_ref[...] + xr_ref[...], 0.0)


def _finalize(aggs, xr):
    bn = 2000
    grid = _N // bn
    nb = _N // bn
    return pl.pallas_call(
        _finalize_body,
        grid=(grid,),
        in_specs=[
            pl.BlockSpec((bn, _D), lambda i: (i, 0)),
            pl.BlockSpec((bn, _D), lambda i, nb=nb: (i + nb, 0)),
            pl.BlockSpec((bn, _D), lambda i: (i, 0)),
        ],
        out_specs=pl.BlockSpec((bn, _D), lambda i: (i, 0)),
        out_shape=jax.ShapeDtypeStruct((_N, _D), jnp.float32),
    )(aggs, aggs, xr)


def kernel(x, edge_index, edge_attr, w_gate, b_gate, W_self, W_root, b_self,
           W_s, W_d, W_a, b_e):
    src = edge_index[0]
    dst = edge_index[1]
    xw, xr, xs, xd = _node_tables(x, W_self, W_root, b_self.reshape(1, _D),
                                  W_s, W_d)
    ea8, gate8 = _edge_tables(edge_attr, W_a, b_e, w_gate, b_gate)
    gate = gate8.reshape(_E)
    aggs, h_e = _make_sc_sparse()(src, dst, gate, xw, xs, xd, ea8)
    h_v = _finalize(aggs, xr)
    return (h_v, edge_index, h_e)


# speedup vs baseline: 1.5488x; 1.5488x over previous
import functools

import jax
import jax.numpy as jnp
from jax import lax
from jax.experimental import pallas as pl
from jax.experimental.pallas import tpu as pltpu
from jax.experimental.pallas import tpu_sc as plsc

_N = 10000
_E = 320000
_D = 128
_DE = 16

_NC = 2
_NS = 16
_EC = _E // _NC
_ET = _EC // _NS
_CH = 40
_NCH = _ET // _CH
_CH8 = _CH // 8
_ZR = (_N // _NS) // 8 * 8


def _node_tables_body(x_ref, ws_ref, wr_ref, bs_ref, wsm_ref, wdm_ref,
                      xw_ref, xr_ref, xs_ref, xd_ref):
    xb = x_ref[...]
    xw_ref[...] = jnp.dot(xb, ws_ref[...], preferred_element_type=jnp.float32)
    xr_ref[...] = jnp.dot(xb, wr_ref[...], preferred_element_type=jnp.float32) + bs_ref[...]
    xs_ref[...] = jnp.dot(xb, wsm_ref[...], preferred_element_type=jnp.float32)
    xd_ref[...] = jnp.dot(xb, wdm_ref[...], preferred_element_type=jnp.float32)


def _node_tables(x, W_self, W_root, b_self2d, W_s, W_d):
    bn = 2000
    grid = _N // bn
    full = lambda shape: pl.BlockSpec(shape, lambda i: (0, 0))
    return pl.pallas_call(
        _node_tables_body,
        grid=(grid,),
        in_specs=[
            pl.BlockSpec((bn, _D), lambda i: (i, 0)),
            full((_D, _D)), full((_D, _D)), full((1, _D)),
            full((_D, _DE)), full((_D, _DE)),
        ],
        out_specs=[
            pl.BlockSpec((bn, _D), lambda i: (i, 0)),
            pl.BlockSpec((bn, _D), lambda i: (i, 0)),
            pl.BlockSpec((bn, _DE), lambda i: (i, 0)),
            pl.BlockSpec((bn, _DE), lambda i: (i, 0)),
        ],
        out_shape=[
            jax.ShapeDtypeStruct((_N, _D), jnp.float32),
            jax.ShapeDtypeStruct((_N, _D), jnp.float32),
            jax.ShapeDtypeStruct((_N, _DE), jnp.float32),
            jax.ShapeDtypeStruct((_N, _DE), jnp.float32),
        ],
    )(x, W_self, W_root, b_self2d, W_s, W_d)


def _edge_tables_body(eaT_ref, waT_ref, be_ref, wgT_ref, bg_ref,
                      eaTo_ref, gate_ref):
    eb = eaT_ref[...]
    eaTo_ref[...] = jnp.dot(waT_ref[...], eb,
                            preferred_element_type=jnp.float32) + be_ref[...]
    z = jnp.dot(wgT_ref[...], eb,
                preferred_element_type=jnp.float32) + bg_ref[...]
    gate_ref[...] = jax.nn.sigmoid(z)


def _edge_tables(edge_attr, W_a, b_e, w_gate, b_gate):
    eaT = edge_attr.T
    waT = W_a.T
    wgT = w_gate.T
    be2 = b_e.reshape(_DE, 1)
    bg2 = b_gate.reshape(1, 1)
    be_cols = 32000
    grid = _E // be_cols
    full = lambda shape: pl.BlockSpec(shape, lambda i: (0, 0))
    eaT_t, gate1 = pl.pallas_call(
        _edge_tables_body,
        grid=(grid,),
        in_specs=[
            pl.BlockSpec((_DE, be_cols), lambda i: (0, i)),
            full((_DE, _DE)), full((_DE, 1)), full((1, _DE)), full((1, 1)),
        ],
        out_specs=[
            pl.BlockSpec((_DE, be_cols), lambda i: (0, i)),
            pl.BlockSpec((1, be_cols), lambda i: (0, i)),
        ],
        out_shape=[
            jax.ShapeDtypeStruct((_DE, _E), jnp.float32),
            jax.ShapeDtypeStruct((1, _E), jnp.float32),
        ],
    )(eaT, waT, be2, wgT, bg2)
    return eaT_t, gate1


@functools.cache
def _make_sc_sparse():
    mesh = plsc.VectorSubcoreMesh(core_axis_name="c", subcore_axis_name="s",
                                  num_cores=_NC, num_subcores=_NS)
    return pl.kernel(
        _sc_sparse_body,
        mesh=mesh,
        out_type=[
            jax.ShapeDtypeStruct((_NC * _N, _D), jnp.float32),
            jax.ShapeDtypeStruct((_DE, _E), jnp.float32),
        ],
        scratch_types=[
            pltpu.VMEM_SHARED((_N, _D), jnp.float32),
            pltpu.VMEM((_ET,), jnp.int32),
            pltpu.VMEM((_ET,), jnp.int32),
            pltpu.VMEM((_CH,), jnp.float32),
            pltpu.VMEM((_CH,), jnp.float32),
            pltpu.VMEM((_CH,), jnp.int32),
            pltpu.VMEM((_CH,), jnp.int32),
            pltpu.VMEM((_CH, _D), jnp.float32),
            pltpu.VMEM((_CH, _D), jnp.float32),
            pltpu.VMEM((_CH, _D), jnp.float32),
            pltpu.VMEM((_CH, _D), jnp.float32),
            pltpu.VMEM((_CH, _DE), jnp.float32),
            pltpu.VMEM((_CH, _DE), jnp.float32),
            pltpu.VMEM((_CH, _DE), jnp.float32),
            pltpu.VMEM((_CH, _DE), jnp.float32),
            pltpu.VMEM((_DE, _CH), jnp.float32),
            pltpu.VMEM((_DE, _CH), jnp.float32),
            pltpu.VMEM((_DE, _CH), jnp.float32),
            pltpu.VMEM((_DE, _CH), jnp.float32),
            pltpu.SemaphoreType.DMA,
            pltpu.SemaphoreType.DMA,
            pltpu.SemaphoreType.DMA,
            pltpu.SemaphoreType.DMA,
            pltpu.SemaphoreType.DMA,
            pltpu.SemaphoreType.DMA,
            pltpu.SemaphoreType.DMA,
        ],
        compiler_params=pltpu.CompilerParams(use_tc_tiling_on_sc=False,
                                            needs_layout_passes=False),
    )


def _sc_sparse_body(src_hbm, dst_hbm, gate_hbm, xw_hbm, xs_hbm, xd_hbm,
                    eaT_hbm, agg_out, heT_out,
                    aggs, srcall, dstall, gv0, gv1, dv0, dv1,
                    rows0, rows1, scat0, scat1, xs0, xs1, xd0, xd1,
                    ea0, ea1, he0, he1,
                    gsem0, gsem1, ssem0, ssem1, hsem0, hsem1, stsem):
    cid = lax.axis_index("c")
    sid = lax.axis_index("s")
    gv = (gv0, gv1)
    dv = (dv0, dv1)
    rows = (rows0, rows1)
    scat = (scat0, scat1)
    xsv = (xs0, xs1)
    xdv = (xd0, xd1)
    eav = (ea0, ea1)
    hev = (he0, he1)
    gsem = (gsem0, gsem1)
    ssem = (ssem0, ssem1)
    hsem = (hsem0, hsem1)
    zero16 = jnp.zeros((16,), jnp.float32)

    tile_base = cid * _EC + sid * _ET
    tile8 = tile_base // 8
    row0 = sid * _ZR

    pltpu.async_copy(src_hbm.at[pl.ds(tile_base, _ET)], srcall, stsem)
    pltpu.async_copy(dst_hbm.at[pl.ds(tile_base, _ET)], dstall, stsem)

    for r in range(_CH):
        for j in range(_D // 16):
            rows0[r, pl.ds(16 * j, 16)] = zero16
    for i in range(_ZR // _CH):
        pltpu.sync_copy(rows0, aggs.at[pl.ds(row0 + i * _CH, _CH)])
    rem = _ZR % _CH
    if rem:
        pltpu.sync_copy(rows0.at[pl.ds(0, rem)],
                        aggs.at[pl.ds(row0 + (_ZR // _CH) * _CH, rem)])
    @pl.when(sid == _NS - 1)
    def _zero_tail():
        pltpu.sync_copy(rows0.at[pl.ds(0, _N - _NS * _ZR)],
                        aggs.at[pl.ds(_NS * _ZR, _N - _NS * _ZR)])
    pltpu.make_async_copy(src_hbm.at[pl.ds(tile_base, _ET)], srcall, stsem).wait()
    pltpu.make_async_copy(dst_hbm.at[pl.ds(tile_base, _ET)], dstall, stsem).wait()
    plsc.subcore_barrier()

    def issue_gathers(b, c):
        base = tile_base + c * _CH
        sidx = srcall.at[pl.ds(c * _CH, _CH)]
        didx = dstall.at[pl.ds(c * _CH, _CH)]
        pltpu.async_copy(xw_hbm.at[sidx], rows[b], gsem[b])
        pltpu.async_copy(xs_hbm.at[sidx], xsv[b], gsem[b])
        pltpu.async_copy(xd_hbm.at[didx], xdv[b], gsem[b])
        pltpu.async_copy(eaT_hbm.at[:, pl.ds(base, _CH)], eav[b], gsem[b])
        pltpu.async_copy(gate_hbm.at[pl.ds(base, _CH)], gv[b], gsem[b])

    def wait_gathers(b, c):
        base = tile_base + c * _CH
        sidx = srcall.at[pl.ds(c * _CH, _CH)]
        didx = dstall.at[pl.ds(c * _CH, _CH)]
        pltpu.make_async_copy(xw_hbm.at[sidx], rows[b], gsem[b]).wait()
        pltpu.make_async_copy(xs_hbm.at[sidx], xsv[b], gsem[b]).wait()
        pltpu.make_async_copy(xd_hbm.at[didx], xdv[b], gsem[b]).wait()
        pltpu.make_async_copy(eaT_hbm.at[:, pl.ds(base, _CH)], eav[b],
                              gsem[b]).wait()
        pltpu.make_async_copy(gate_hbm.at[pl.ds(base, _CH)], gv[b], gsem[b]).wait()

    def wait_scatter(b, c):
        pltpu.make_async_copy(scat[b], aggs.at[dv[b]], ssem[b]).wait()

    def wait_hestore(b, c):
        pltpu.make_async_copy(hev[b],
                              heT_out.at[:, pl.ds(tile_base + c * _CH, _CH)],
                              hsem[b]).wait()

    def process(b, c, first):
        wait_gathers(b, c)
        if not first:
            wait_scatter(b, c)
            wait_hestore(b, c)
        dv[b][pl.ds(0, 16)] = dstall[pl.ds(c * _CH, 16)]
        dv[b][pl.ds(16, 16)] = dstall[pl.ds(c * _CH + 16, 16)]
        dv[b][pl.ds(24, 16)] = dstall[pl.ds(c * _CH + 24, 16)]
        g0 = gv[b][pl.ds(0, 16)]
        g1 = gv[b][pl.ds(16, 16)]
        g2 = gv[b][pl.ds(24, 16)]
        for e in range(_CH):
            if e < 16:
                g = g0[e]
            elif e < 32:
                g = g1[e - 16]
            else:
                g = g2[e - 24]
            for j in range(_D // 16):
                sl = pl.ds(16 * j, 16)
                scat[b][e, sl] = rows[b][e, sl] * g
        pltpu.async_copy(scat[b], aggs.at[dv[b]], ssem[b], add=True)
        rowi = lax.iota(jnp.int32, 16)
        for e in range(_CH):
            coli = jnp.full((16,), e, jnp.int32)
            v = (xsv[b][e, :] + xdv[b][e, :]
                 + plsc.load_gather(eav[b], (rowi, coli)))
            plsc.store_scatter(hev[b], (rowi, coli),
                               jnp.maximum(v, 0.0))
        pltpu.async_copy(hev[b],
                         heT_out.at[:, pl.ds(tile_base + c * _CH, _CH)],
                         hsem[b])

    issue_gathers(0, 0)
    issue_gathers(1, 1)

    def pair_body(i, carry):
        c0 = 2 * i

        @pl.when(i == 0)
        def _first():
            process(0, c0, True)
            issue_gathers(0, c0 + 2)
            process(1, c0 + 1, True)
            issue_gathers(1, c0 + 3)

        @pl.when(i > 0)
        def _steady():
            process(0, c0, False)
            issue_gathers(0, c0 + 2)
            process(1, c0 + 1, False)
            issue_gathers(1, c0 + 3)

        return carry

    lax.fori_loop(0, _NCH // 2 - 1, pair_body, 0)
    process(0, _NCH - 2, False)
    process(1, _NCH - 1, False)
    wait_scatter(0, _NCH - 2)
    wait_hestore(0, _NCH - 2)
    wait_scatter(1, _NCH - 1)
    wait_hestore(1, _NCH - 1)

    plsc.subcore_barrier()
    pltpu.sync_copy(aggs.at[pl.ds(row0, _ZR)],
                    agg_out.at[pl.ds(cid * _N + row0, _ZR)])
    @pl.when(sid == _NS - 1)
    def _flush_tail():
        pltpu.sync_copy(aggs.at[pl.ds(_NS * _ZR, _N - _NS * _ZR)],
                        agg_out.at[pl.ds(cid * _N + _NS * _ZR, _N - _NS * _ZR)])


def _finalize_body(a0_ref, a1_ref, xr_ref, hv_ref):
    hv_ref[...] = jnp.maximum(a0_ref[...] + a1_ref[...] + xr_ref[...], 0.0)


def _finalize(aggs, xr):
    bn = 2000
    grid = _N // bn
    nb = _N // bn
    return pl.pallas_call(
        _finalize_body,
        grid=(grid,),
        in_specs=[
            pl.BlockSpec((bn, _D), lambda i: (i, 0)),
            pl.BlockSpec((bn, _D), lambda i, nb=nb: (i + nb, 0)),
            pl.BlockSpec((bn, _D), lambda i: (i, 0)),
        ],
        out_specs=pl.BlockSpec((bn, _D), lambda i: (i, 0)),
        out_shape=jax.ShapeDtypeStruct((_N, _D), jnp.float32),
    )(aggs, aggs, xr)


def kernel(x, edge_index, edge_attr, w_gate, b_gate, W_self, W_root, b_self,
           W_s, W_d, W_a, b_e):
    src = edge_index[0]
    dst = edge_index[1]
    xw, xr, xs, xd = _node_tables(x, W_self, W_root, b_self.reshape(1, _D),
                                  W_s, W_d)
    eaT_t, gate1 = _edge_tables(edge_attr, W_a, b_e, w_gate, b_gate)
    gate = gate1.reshape(_E)
    aggs, heT = _make_sc_sparse()(src, dst, gate, xw, xs, xd, eaT_t)
    h_v = _finalize(aggs, xr)
    return (h_v, edge_index, heT.T)


# speedup vs baseline: 1.5631x; 1.0093x over previous
import functools

import jax
import jax.numpy as jnp
import numpy as np
from jax import lax
from jax.experimental import pallas as pl
from jax.experimental.pallas import tpu as pltpu
from jax.experimental.pallas import tpu_sc as plsc

_N = 10000
_E = 320000
_D = 128
_DE = 16

_NC = 2
_NS = 16
_EC = _E // _NC
_ET = _EC // _NS
_CH = 40
_NCH = _ET // _CH
_CH8 = _CH // 8
_ZR = (_N // _NS) // 8 * 8

_PERM = np.zeros((128, 128), np.float32)
for _g in range(4):
    for _h in range(2):
        for _l in range(16):
            _PERM[32 * _g + 16 * _h + _l, 32 * _g + 2 * _l + _h] = 1.0


def _node_tables_body(x_ref, ws_ref, wr_ref, bs_ref, wsm_ref, wdm_ref,
                      xw_ref, xr_ref, xs_ref, xd_ref):
    xb = x_ref[...]
    xw_ref[...] = jnp.dot(xb, ws_ref[...],
                          preferred_element_type=jnp.float32).astype(jnp.bfloat16)
    xr_ref[...] = jnp.dot(xb, wr_ref[...], preferred_element_type=jnp.float32) + bs_ref[...]
    xs_ref[...] = jnp.dot(xb, wsm_ref[...], preferred_element_type=jnp.float32)
    xd_ref[...] = jnp.dot(xb, wdm_ref[...], preferred_element_type=jnp.float32)


def _node_tables(x, W_self, W_root, b_self2d, W_s, W_d):
    bn = 2000
    grid = _N // bn
    full = lambda shape: pl.BlockSpec(shape, lambda i: (0, 0))
    return pl.pallas_call(
        _node_tables_body,
        grid=(grid,),
        in_specs=[
            pl.BlockSpec((bn, _D), lambda i: (i, 0)),
            full((_D, _D)), full((_D, _D)), full((1, _D)),
            full((_D, _DE)), full((_D, _DE)),
        ],
        out_specs=[
            pl.BlockSpec((bn, _D), lambda i: (i, 0)),
            pl.BlockSpec((bn, _D), lambda i: (i, 0)),
            pl.BlockSpec((bn, _DE), lambda i: (i, 0)),
            pl.BlockSpec((bn, _DE), lambda i: (i, 0)),
        ],
        out_shape=[
            jax.ShapeDtypeStruct((_N, _D), jnp.bfloat16),
            jax.ShapeDtypeStruct((_N, _D), jnp.float32),
            jax.ShapeDtypeStruct((_N, _DE), jnp.float32),
            jax.ShapeDtypeStruct((_N, _DE), jnp.float32),
        ],
    )(x, W_self, W_root, b_self2d, W_s, W_d)


def _edge_tables_body(eaT_ref, waT_ref, be_ref, wgT_ref, bg_ref,
                      eaTo_ref, gate_ref):
    eb = eaT_ref[...]
    eaTo_ref[...] = jnp.dot(waT_ref[...], eb,
                            preferred_element_type=jnp.float32) + be_ref[...]
    z = jnp.dot(wgT_ref[...], eb,
                preferred_element_type=jnp.float32) + bg_ref[...]
    gate_ref[...] = jax.nn.sigmoid(z)


def _edge_tables(edge_attr, W_a, b_e, w_gate, b_gate):
    eaT = edge_attr.T
    waT = W_a.T
    wgT = w_gate.T
    be2 = b_e.reshape(_DE, 1)
    bg2 = b_gate.reshape(1, 1)
    be_cols = 32000
    grid = _E // be_cols
    full = lambda shape: pl.BlockSpec(shape, lambda i: (0, 0))
    eaT_t, gate1 = pl.pallas_call(
        _edge_tables_body,
        grid=(grid,),
        in_specs=[
            pl.BlockSpec((_DE, be_cols), lambda i: (0, i)),
            full((_DE, _DE)), full((_DE, 1)), full((1, _DE)), full((1, 1)),
        ],
        out_specs=[
            pl.BlockSpec((_DE, be_cols), lambda i: (0, i)),
            pl.BlockSpec((1, be_cols), lambda i: (0, i)),
        ],
        out_shape=[
            jax.ShapeDtypeStruct((_DE, _E), jnp.float32),
            jax.ShapeDtypeStruct((1, _E), jnp.float32),
        ],
    )(eaT, waT, be2, wgT, bg2)
    return eaT_t, gate1


@functools.cache
def _make_sc_sparse():
    mesh = plsc.VectorSubcoreMesh(core_axis_name="c", subcore_axis_name="s",
                                  num_cores=_NC, num_subcores=_NS)
    return pl.kernel(
        _sc_sparse_body,
        mesh=mesh,
        out_type=[
            jax.ShapeDtypeStruct((_NC * _N, _D), jnp.float32),
            jax.ShapeDtypeStruct((_DE, _E), jnp.float32),
        ],
        scratch_types=[
            pltpu.VMEM_SHARED((_N, _D), jnp.float32),
            pltpu.VMEM((_ET,), jnp.int32),
            pltpu.VMEM((_ET,), jnp.int32),
            pltpu.VMEM((_CH,), jnp.float32),
            pltpu.VMEM((_CH,), jnp.float32),
            pltpu.VMEM((_CH,), jnp.int32),
            pltpu.VMEM((_CH,), jnp.int32),
            pltpu.VMEM((_CH, _D), jnp.bfloat16),
            pltpu.VMEM((_CH, _D), jnp.bfloat16),
            pltpu.VMEM((_CH, _D), jnp.float32),
            pltpu.VMEM((_CH, _D), jnp.float32),
            pltpu.VMEM((_CH, _DE), jnp.float32),
            pltpu.VMEM((_CH, _DE), jnp.float32),
            pltpu.VMEM((_CH, _DE), jnp.float32),
            pltpu.VMEM((_CH, _DE), jnp.float32),
            pltpu.VMEM((_DE, _CH), jnp.float32),
            pltpu.VMEM((_DE, _CH), jnp.float32),
            pltpu.VMEM((_DE, _CH), jnp.float32),
            pltpu.VMEM((_DE, _CH), jnp.float32),
            pltpu.SemaphoreType.DMA,
            pltpu.SemaphoreType.DMA,
            pltpu.SemaphoreType.DMA,
            pltpu.SemaphoreType.DMA,
            pltpu.SemaphoreType.DMA,
            pltpu.SemaphoreType.DMA,
            pltpu.SemaphoreType.DMA,
        ],
        compiler_params=pltpu.CompilerParams(use_tc_tiling_on_sc=False,
                                            needs_layout_passes=False),
    )


def _sc_sparse_body(src_hbm, dst_hbm, gate_hbm, xw_hbm, xs_hbm, xd_hbm,
                    eaT_hbm, agg_out, heT_out,
                    aggs, srcall, dstall, gv0, gv1, dv0, dv1,
                    rows0, rows1, scat0, scat1, xs0, xs1, xd0, xd1,
                    ea0, ea1, he0, he1,
                    gsem0, gsem1, ssem0, ssem1, hsem0, hsem1, stsem):
    cid = lax.axis_index("c")
    sid = lax.axis_index("s")
    gv = (gv0, gv1)
    dv = (dv0, dv1)
    rows = (rows0, rows1)
    scat = (scat0, scat1)
    xsv = (xs0, xs1)
    xdv = (xd0, xd1)
    eav = (ea0, ea1)
    hev = (he0, he1)
    gsem = (gsem0, gsem1)
    ssem = (ssem0, ssem1)
    hsem = (hsem0, hsem1)
    zero16 = jnp.zeros((16,), jnp.float32)

    tile_base = cid * _EC + sid * _ET
    tile8 = tile_base // 8
    row0 = sid * _ZR

    pltpu.async_copy(src_hbm.at[pl.ds(tile_base, _ET)], srcall, stsem)
    pltpu.async_copy(dst_hbm.at[pl.ds(tile_base, _ET)], dstall, stsem)

    for r in range(_CH):
        for j in range(_D // 16):
            scat0[r, pl.ds(16 * j, 16)] = zero16
    for i in range(_ZR // _CH):
        pltpu.sync_copy(scat0, aggs.at[pl.ds(row0 + i * _CH, _CH)])
    rem = _ZR % _CH
    if rem:
        pltpu.sync_copy(scat0.at[pl.ds(0, rem)],
                        aggs.at[pl.ds(row0 + (_ZR // _CH) * _CH, rem)])
    @pl.when(sid == _NS - 1)
    def _zero_tail():
        pltpu.sync_copy(scat0.at[pl.ds(0, _N - _NS * _ZR)],
                        aggs.at[pl.ds(_NS * _ZR, _N - _NS * _ZR)])
    pltpu.make_async_copy(src_hbm.at[pl.ds(tile_base, _ET)], srcall, stsem).wait()
    pltpu.make_async_copy(dst_hbm.at[pl.ds(tile_base, _ET)], dstall, stsem).wait()
    plsc.subcore_barrier()

    def issue_gathers(b, c):
        base = tile_base + c * _CH
        sidx = srcall.at[pl.ds(c * _CH, _CH)]
        didx = dstall.at[pl.ds(c * _CH, _CH)]
        pltpu.async_copy(xw_hbm.at[sidx], rows[b], gsem[b])
        pltpu.async_copy(xs_hbm.at[sidx], xsv[b], gsem[b])
        pltpu.async_copy(xd_hbm.at[didx], xdv[b], gsem[b])
        pltpu.async_copy(eaT_hbm.at[:, pl.ds(base, _CH)], eav[b], gsem[b])
        pltpu.async_copy(gate_hbm.at[pl.ds(base, _CH)], gv[b], gsem[b])

    def wait_gathers(b, c):
        base = tile_base + c * _CH
        sidx = srcall.at[pl.ds(c * _CH, _CH)]
        didx = dstall.at[pl.ds(c * _CH, _CH)]
        pltpu.make_async_copy(xw_hbm.at[sidx], rows[b], gsem[b]).wait()
        pltpu.make_async_copy(xs_hbm.at[sidx], xsv[b], gsem[b]).wait()
        pltpu.make_async_copy(xd_hbm.at[didx], xdv[b], gsem[b]).wait()
        pltpu.make_async_copy(eaT_hbm.at[:, pl.ds(base, _CH)], eav[b],
                              gsem[b]).wait()
        pltpu.make_async_copy(gate_hbm.at[pl.ds(base, _CH)], gv[b], gsem[b]).wait()

    def wait_scatter(b, c):
        pltpu.make_async_copy(scat[b], aggs.at[dv[b]], ssem[b]).wait()

    def wait_hestore(b, c):
        pltpu.make_async_copy(hev[b],
                              heT_out.at[:, pl.ds(tile_base + c * _CH, _CH)],
                              hsem[b]).wait()

    def process(b, c, first):
        wait_gathers(b, c)
        if not first:
            wait_scatter(b, c)
            wait_hestore(b, c)
        dv[b][pl.ds(0, 16)] = dstall[pl.ds(c * _CH, 16)]
        dv[b][pl.ds(16, 16)] = dstall[pl.ds(c * _CH + 16, 16)]
        dv[b][pl.ds(24, 16)] = dstall[pl.ds(c * _CH + 24, 16)]
        g0 = gv[b][pl.ds(0, 16)]
        g1 = gv[b][pl.ds(16, 16)]
        g2 = gv[b][pl.ds(24, 16)]
        for e in range(_CH):
            if e < 16:
                g = g0[e]
            elif e < 32:
                g = g1[e - 16]
            else:
                g = g2[e - 24]
            for j in range(_D // 32):
                v = rows[b][e, pl.ds(32 * j, 32)]
                lo, hi = plsc.unpack(v, format=plsc.PackFormat.INTERLEAVED)
                scat[b][e, pl.ds(32 * j, 16)] = lo * g
                scat[b][e, pl.ds(32 * j + 16, 16)] = hi * g
        pltpu.async_copy(scat[b], aggs.at[dv[b]], ssem[b], add=True)
        rowi = lax.iota(jnp.int32, 16)
        for e in range(_CH):
            coli = jnp.full((16,), e, jnp.int32)
            v = (xsv[b][e, :] + xdv[b][e, :]
                 + plsc.load_gather(eav[b], (rowi, coli)))
            plsc.store_scatter(hev[b], (rowi, coli),
                               jnp.maximum(v, 0.0))
        pltpu.async_copy(hev[b],
                         heT_out.at[:, pl.ds(tile_base + c * _CH, _CH)],
                         hsem[b])

    issue_gathers(0, 0)
    issue_gathers(1, 1)

    def pair_body(i, carry):
        c0 = 2 * i

        @pl.when(i == 0)
        def _first():
            process(0, c0, True)
            issue_gathers(0, c0 + 2)
            process(1, c0 + 1, True)
            issue_gathers(1, c0 + 3)

        @pl.when(i > 0)
        def _steady():
            process(0, c0, False)
            issue_gathers(0, c0 + 2)
            process(1, c0 + 1, False)
            issue_gathers(1, c0 + 3)

        return carry

    lax.fori_loop(0, _NCH // 2 - 1, pair_body, 0)
    process(0, _NCH - 2, False)
    process(1, _NCH - 1, False)
    wait_scatter(0, _NCH - 2)
    wait_hestore(0, _NCH - 2)
    wait_scatter(1, _NCH - 1)
    wait_hestore(1, _NCH - 1)

    plsc.subcore_barrier()
    pltpu.sync_copy(aggs.at[pl.ds(row0, _ZR)],
                    agg_out.at[pl.ds(cid * _N + row0, _ZR)])
    @pl.when(sid == _NS - 1)
    def _flush_tail():
        pltpu.sync_copy(aggs.at[pl.ds(_NS * _ZR, _N - _NS * _ZR)],
                        agg_out.at[pl.ds(cid * _N + _NS * _ZR, _N - _NS * _ZR)])


def _finalize_body(a0_ref, a1_ref, xr_ref, hv_ref):
    hv_ref[...] = jnp.maximum(a0_ref[...] + a1_ref[...] + xr_ref[...], 0.0)


def _finalize(aggs, xr):
    bn = 2000
    grid = _N // bn
    nb = _N // bn
    return pl.pallas_call(
        _finalize_body,
        grid=(grid,),
        in_specs=[
            pl.BlockSpec((bn, _D), lambda i: (i, 0)),
            pl.BlockSpec((bn, _D), lambda i, nb=nb: (i + nb, 0)),
            pl.BlockSpec((bn, _D), lambda i: (i, 0)),
        ],
        out_specs=pl.BlockSpec((bn, _D), lambda i: (i, 0)),
        out_shape=jax.ShapeDtypeStruct((_N, _D), jnp.float32),
    )(aggs, aggs, xr)


def kernel(x, edge_index, edge_attr, w_gate, b_gate, W_self, W_root, b_self,
           W_s, W_d, W_a, b_e):
    src = edge_index[0]
    dst = edge_index[1]
    xw, xr, xs, xd = _node_tables(x, W_self @ jnp.asarray(_PERM), W_root,
                                  b_self.reshape(1, _D), W_s, W_d)
    eaT_t, gate1 = _edge_tables(edge_attr, W_a, b_e, w_gate, b_gate)
    gate = gate1.reshape(_E)
    aggs, heT = _make_sc_sparse()(src, dst, gate, xw, xs, xd, eaT_t)
    h_v = _finalize(aggs, xr)
    return (h_v, edge_index, heT.T)
